# baseline (speedup 1.0000x reference)
import functools
import jax
import jax.numpy as jnp
from jax import lax
from jax.experimental import pallas as pl
from jax.experimental.pallas import tpu as pltpu
from jax.experimental.pallas import tpu_sc as plsc

D_MODEL = 4096
N_HEADS = 64
TOP_K = 8
BLK = 1024
N_TOK = 16384
GRP = 16


def _matmul_body(x_ref, w_ref, b_ref, logits_ref):
    logits_ref[...] = jax.lax.dot_general(
        w_ref[...], x_ref[...], (((1,), (1,)), ((), ())),
        preferred_element_type=jnp.float32,
        precision=jax.lax.Precision.DEFAULT,
    ) + b_ref[...]


def _tc_logits(x2, W, b2):
    n_tok = x2.shape[0]
    grid = (n_tok // BLK,)
    return pl.pallas_call(
        _matmul_body,
        grid=grid,
        in_specs=[
            pl.BlockSpec((BLK, D_MODEL), lambda i: (i, 0)),
            pl.BlockSpec((N_HEADS, D_MODEL), lambda i: (0, 0)),
            pl.BlockSpec((N_HEADS, 1), lambda i: (0, 0)),
        ],
        out_specs=pl.BlockSpec((N_HEADS, BLK), lambda i: (0, i)),
        out_shape=jax.ShapeDtypeStruct((N_HEADS, n_tok), jnp.float32),
        compiler_params=pltpu.CompilerParams(
            dimension_semantics=("parallel",)),
    )(x2, W, b2)


def _sc_topk_kernel(n_tok):
    info = plsc.get_sparse_core_info()
    nc, ns = info.num_cores, info.num_subcores
    nw = nc * ns
    chunk = n_tok // nw
    ngrp = chunk // GRP
    mesh = plsc.VectorSubcoreMesh(core_axis_name="c", subcore_axis_name="s")

    @functools.partial(
        pl.kernel, mesh=mesh,
        out_type=[
            jax.ShapeDtypeStruct((TOP_K, n_tok), jnp.float32),
            jax.ShapeDtypeStruct((TOP_K, n_tok), jnp.int32),
        ],
        scratch_types=[
            pltpu.VMEM((N_HEADS, chunk), jnp.float32),
            pltpu.VMEM((TOP_K, chunk), jnp.float32),
            pltpu.VMEM((TOP_K, chunk), jnp.int32),
        ],
    )
    def k(logits_hbm, gates_hbm, idx_hbm, lg_v, gates_v, idx_v):
        wid = lax.axis_index("s") * nc + lax.axis_index("c")
        base = wid * chunk
        pltpu.sync_copy(logits_hbm.at[:, pl.ds(base, chunk)], lg_v)

        def body(g, _):
            off = g * GRP
            heads = [lg_v[h, pl.ds(off, GRP)] for h in range(N_HEADS)]
            alive = heads
            neg = jnp.full((GRP,), -jnp.inf, jnp.float32)
            for k_i in range(TOP_K):
                t = alive
                while len(t) > 1:
                    t = [jnp.maximum(t[i], t[i + 1]) for i in range(0, len(t) - 1, 2)] \
                        + ([t[-1]] if len(t) % 2 else [])
                m = t[0]
                am = jnp.full((GRP,), 64.0, jnp.float32)
                for h in range(N_HEADS - 1, -1, -1):
                    am = jnp.where(alive[h] == m, jnp.float32(h), am)
                gates_v[k_i, pl.ds(off, GRP)] = m
                idx_v[k_i, pl.ds(off, GRP)] = am.astype(jnp.int32)
                if k_i + 1 < TOP_K:
                    alive = [jnp.where(alive[h] == m, neg, alive[h])
                             for h in range(N_HEADS)]
            top = [gates_v[k_i, pl.ds(off, GRP)] for k_i in range(TOP_K)]
            es = [jnp.exp(v - top[0]) for v in top]
            tot = es[0]
            for e in es[1:]:
                tot = tot + e
            for k_i in range(TOP_K):
                gates_v[k_i, pl.ds(off, GRP)] = es[k_i] / tot
            return _

        lax.fori_loop(0, ngrp, body, 0)
        pltpu.sync_copy(gates_v, gates_hbm.at[:, pl.ds(base, chunk)])
        pltpu.sync_copy(idx_v, idx_hbm.at[:, pl.ds(base, chunk)])

    return k


def kernel(x, W, b):
    B, T, D = x.shape
    n_tok = B * T
    x2 = x.reshape(n_tok, D)
    b2 = b.reshape(N_HEADS, 1)
    logits_t = _tc_logits(x2, W, b2)
    gates_t, idx_t = _sc_topk_kernel(n_tok)(logits_t)
    gates = gates_t.T.reshape(B, T, TOP_K)
    idx = idx_t.T.reshape(B, T, TOP_K)
    return gates, idx


# speedup vs baseline: 1.4773x; 1.4773x over previous
import jax
import jax.numpy as jnp
from jax.experimental import pallas as pl
from jax.experimental.pallas import tpu as pltpu

D_MODEL = 4096
N_HEADS = 64
TOP_K = 8
BLK = 1024


def _router_body(x_ref, w_ref, b_ref, gates_ref, idx_ref):
    x = x_ref[...]
    w = w_ref[...]
    logits = jax.lax.dot_general(
        w, x, (((1,), (1,)), ((), ())),
        preferred_element_type=jnp.float32,
        precision=jax.lax.Precision.DEFAULT,
    )
    logits = logits + b_ref[...]

    iota_f = jax.lax.broadcasted_iota(jnp.int32, logits.shape, 0).astype(jnp.float32)
    cur = logits
    vals = []
    idxs = []
    for k in range(TOP_K):
        m = jnp.max(cur, axis=0, keepdims=True)
        eq = cur == m
        am = jnp.min(jnp.where(eq, iota_f, 64.0), axis=0, keepdims=True)
        vals.append(m)
        idxs.append(am)
        if k + 1 < TOP_K:
            cur = jnp.where(eq, -jnp.inf, cur)
    topv = jnp.concatenate(vals, axis=0)
    topi = jnp.concatenate(idxs, axis=0)

    e = jnp.exp(topv - topv[:1])
    gates_ref[...] = e / jnp.sum(e, axis=0, keepdims=True)
    idx_ref[...] = topi.astype(jnp.int32)


def kernel(x, W, b):
    B, T, D = x.shape
    n_tok = B * T
    x2 = x.reshape(n_tok, D)
    b2 = b.reshape(N_HEADS, 1)
    grid = (n_tok // BLK,)
    gates_t, idx_t = pl.pallas_call(
        _router_body,
        grid=grid,
        in_specs=[
            pl.BlockSpec((BLK, D), lambda i: (i, 0)),
            pl.BlockSpec((N_HEADS, D), lambda i: (0, 0)),
            pl.BlockSpec((N_HEADS, 1), lambda i: (0, 0)),
        ],
        out_specs=[
            pl.BlockSpec((TOP_K, BLK), lambda i: (0, i)),
            pl.BlockSpec((TOP_K, BLK), lambda i: (0, i)),
        ],
        out_shape=[
            jax.ShapeDtypeStruct((TOP_K, n_tok), jnp.float32),
            jax.ShapeDtypeStruct((TOP_K, n_tok), jnp.int32),
        ],
        compiler_params=pltpu.CompilerParams(dimension_semantics=("parallel",)),
    )(x2, W, b2)
    gates = gates_t.T.reshape(B, T, TOP_K)
    idx = idx_t.T.reshape(B, T, TOP_K)
    return gates, idx
